# baseline (speedup 1.0000x reference)
import functools

import jax
import jax.numpy as jnp
from jax.experimental import pallas as pl
from jax.experimental.pallas import tpu as pltpu

_BIG = 1e20
_NC = 4


def _authpct_body(real_ref, gen_ref, out_ref,
                  rs_ref, rncol_ref, rnrow_ref, gnrow_ref,
                  min2_ref, rmin_ref, bkey_ref, bval_ref, *, blk, nblk, n):
    ph = pl.program_id(0)
    p = pl.program_id(1)
    q = pl.program_id(2)

    @pl.when((ph == 0) & (p == 0) & (q == 0))
    def _init():
        r = real_ref[...]
        g = gen_ref[...]
        rs_ref[...] = r * -2.0
        rn = jnp.sum(r * r, axis=1, keepdims=True)
        rncol_ref[...] = rn
        rnrow_ref[...] = rn.T
        gnrow_ref[...] = jnp.sum(g * g, axis=1, keepdims=True).T
        min2_ref[...] = jnp.full((1, n), _BIG, jnp.float32)
        bkey_ref[...] = jnp.full((1, n), _BIG, jnp.float32)
        bval_ref[...] = jnp.zeros((1, n), jnp.float32)

    cb = blk // _NC

    @pl.when((ph == 0) & (p <= q))
    def _real_real():

        @pl.when(p == q)
        def _diag():
            for si in range(_NC):
                asub = rs_ref[pl.ds(p * blk + si * cb, cb), :]
                a2s = rncol_ref[pl.ds(p * blk + si * cb, cb), :]
                a2sr = rnrow_ref[0:1, pl.ds(p * blk + si * cb, cb)]
                for sj in range(si, _NC):
                    b = real_ref[pl.ds(q * blk + sj * cb, cb), :]
                    prod = jax.lax.dot_general(
                        asub, b, (((1,), (1,)), ((), ())),
                        preferred_element_type=jnp.float32)
                    sl = pl.ds(q * blk + sj * cb, cb)
                    b2c = rnrow_ref[0:1, sl]
                    if si == sj:
                        rows = jax.lax.broadcasted_iota(jnp.int32, (cb, cb), 0)
                        cols = jax.lax.broadcasted_iota(jnp.int32, (cb, cb), 1)
                        prod = jnp.where(rows == cols, _BIG, prod)
                    colmin = jnp.min(prod + a2s, axis=0, keepdims=True)
                    min2_ref[0:1, sl] = jnp.minimum(
                        min2_ref[0:1, sl], jnp.maximum(colmin + b2c, 0.0))
                    if si != sj:
                        rowmin = jnp.min(prod + b2c, axis=1, keepdims=True)
                        sl_i = pl.ds(p * blk + si * cb, cb)
                        min2_ref[0:1, sl_i] = jnp.minimum(
                            min2_ref[0:1, sl_i],
                            jnp.maximum(rowmin.T + a2sr, 0.0))

        @pl.when(p != q)
        def _offdiag():
            a = rs_ref[pl.ds(p * blk, blk), :]
            a2 = rncol_ref[pl.ds(p * blk, blk), :]
            a2r = rnrow_ref[0:1, pl.ds(p * blk, blk)]
            rowmins = []
            for ci in range(_NC):
                b = real_ref[pl.ds(q * blk + ci * cb, cb), :]
                prod = jax.lax.dot_general(a, b, (((1,), (1,)), ((), ())),
                                           preferred_element_type=jnp.float32)
                sl = pl.ds(q * blk + ci * cb, cb)
                b2c = rnrow_ref[0:1, sl]
                colmin = jnp.min(prod + a2, axis=0, keepdims=True)
                min2_ref[0:1, sl] = jnp.minimum(
                    min2_ref[0:1, sl], jnp.maximum(colmin + b2c, 0.0))
                rowmins.append(jnp.min(prod + b2c, axis=1, keepdims=True))
            rowmin = rowmins[0]
            for rm in rowmins[1:]:
                rowmin = jnp.minimum(rowmin, rm)
            sl_p = pl.ds(p * blk, blk)
            min2_ref[0:1, sl_p] = jnp.minimum(min2_ref[0:1, sl_p],
                                              jnp.maximum(rowmin.T + a2r, 0.0))

    @pl.when((ph == 1) & (p == 0) & (q == 0))
    def _sqrt_real_mins():
        rmin_ref[...] = jnp.sqrt(min2_ref[...])

    @pl.when(ph == 1)
    def _real_gen():
        a = rs_ref[pl.ds(q * blk, blk), :]
        a2 = rncol_ref[pl.ds(q * blk, blk), :]
        rv = rmin_ref[0:1, pl.ds(q * blk, blk)]
        rvt = rv.T
        for ci in range(_NC):
            g = gen_ref[pl.ds(p * blk + ci * cb, cb), :]
            prod = jax.lax.dot_general(a, g, (((1,), (1,)), ((), ())),
                                       preferred_element_type=jnp.float32)
            t = prod + a2
            lk = jnp.min(t, axis=0, keepdims=True)
            cand = jnp.where(t == lk, rvt, _BIG)
            lv = jnp.min(cand, axis=0, keepdims=True)
            sl = pl.ds(p * blk + ci * cb, cb)
            ck = bkey_ref[0:1, sl]
            cv = bval_ref[0:1, sl]
            upd = lk < ck
            bkey_ref[0:1, sl] = jnp.where(upd, lk, ck)
            bval_ref[0:1, sl] = jnp.where(upd, lv, cv)

    @pl.when((ph == 1) & (p == nblk - 1) & (q == nblk - 1))
    def _finalize():
        g2r = gnrow_ref[...]
        d1 = jnp.sqrt(jnp.maximum(bkey_ref[...] + g2r, 0.0))
        d2v = bval_ref[...]
        authen = jax.nn.sigmoid((d2v - d1) * 10.0)
        out_ref[...] = (-100.0 * (jnp.sum(authen) / n)).reshape(1, 1)


@jax.jit
def kernel(real_stats, gen_stats):
    real_stats = jax.lax.stop_gradient(real_stats)
    n, k = real_stats.shape
    assert gen_stats.shape[0] == n
    blk = 1024 if n % 1024 == 0 else max(b for b in (512, 256, 128, 8)
                                         if n % b == 0)
    nblk = n // blk
    out = pl.pallas_call(
        functools.partial(_authpct_body, blk=blk, nblk=nblk, n=n),
        grid=(2, nblk, nblk),
        in_specs=[
            pl.BlockSpec((n, k), lambda ph, p, q: (0, 0)),
            pl.BlockSpec((n, k), lambda ph, p, q: (0, 0)),
        ],
        out_specs=pl.BlockSpec((1, 1), lambda ph, p, q: (0, 0)),
        out_shape=jax.ShapeDtypeStruct((1, 1), jnp.float32),
        scratch_shapes=[
            pltpu.VMEM((n, k), jnp.float32),
            pltpu.VMEM((n, 1), jnp.float32),
            pltpu.VMEM((1, n), jnp.float32),
            pltpu.VMEM((1, n), jnp.float32),
            pltpu.VMEM((1, n), jnp.float32),
            pltpu.VMEM((1, n), jnp.float32),
            pltpu.VMEM((1, n), jnp.float32),
            pltpu.VMEM((1, n), jnp.float32),
        ],
        compiler_params=pltpu.CompilerParams(
            dimension_semantics=("arbitrary", "arbitrary", "arbitrary"),
        ),
    )(real_stats, gen_stats)
    return out[0, 0]


# speedup vs baseline: 1.0136x; 1.0136x over previous
import functools

import jax
import jax.numpy as jnp
from jax.experimental import pallas as pl
from jax.experimental.pallas import tpu as pltpu

_BIG = 1e20
_NC = 4


def _authpct_body(real_ref, gen_ref, out_ref,
                  rs_ref, rncol_ref, rnrow_ref, gnrow_ref,
                  min2_ref, rmin_ref, bkey_ref, bval_ref, *, blk, nblk, n):
    ph = pl.program_id(0)
    p = pl.program_id(1)
    q = pl.program_id(2)

    @pl.when((ph == 0) & (p == 0) & (q == 0))
    def _init():
        r = real_ref[...]
        g = gen_ref[...]
        rs_ref[...] = r * -2.0
        rn = jnp.sum(r * r, axis=1, keepdims=True)
        rncol_ref[...] = rn
        rnrow_ref[...] = rn.T
        gnrow_ref[...] = jnp.sum(g * g, axis=1, keepdims=True).T
        min2_ref[...] = jnp.full((1, n), _BIG, jnp.float32)
        bkey_ref[...] = jnp.full((1, n), _BIG, jnp.float32)
        bval_ref[...] = jnp.zeros((1, n), jnp.float32)

    cb = blk // _NC

    @pl.when((ph == 0) & (p <= q))
    def _real_real():
        a = rs_ref[pl.ds(p * blk, blk), :]
        a2 = rncol_ref[pl.ds(p * blk, blk), :]
        a2r = rnrow_ref[0:1, pl.ds(p * blk, blk)]

        @pl.when(p == q)
        def _diag():
            for ci in range(_NC):
                b = real_ref[pl.ds(q * blk + ci * cb, cb), :]
                prod = jax.lax.dot_general(a, b, (((1,), (1,)), ((), ())),
                                           preferred_element_type=jnp.float32)
                rows = jax.lax.broadcasted_iota(jnp.int32, (blk, cb), 0)
                cols = jax.lax.broadcasted_iota(jnp.int32, (blk, cb), 1)
                masked = jnp.where(rows == cols + ci * cb, _BIG, prod)
                colmin = jnp.min(masked + a2, axis=0, keepdims=True)
                sl = pl.ds(q * blk + ci * cb, cb)
                b2c = rnrow_ref[0:1, sl]
                min2_ref[0:1, sl] = jnp.minimum(
                    min2_ref[0:1, sl], jnp.maximum(colmin + b2c, 0.0))

        @pl.when(p != q)
        def _offdiag():
            rowmins = []
            for ci in range(_NC):
                b = real_ref[pl.ds(q * blk + ci * cb, cb), :]
                prod = jax.lax.dot_general(a, b, (((1,), (1,)), ((), ())),
                                           preferred_element_type=jnp.float32)
                sl = pl.ds(q * blk + ci * cb, cb)
                b2c = rnrow_ref[0:1, sl]
                colmin = jnp.min(prod + a2, axis=0, keepdims=True)
                min2_ref[0:1, sl] = jnp.minimum(
                    min2_ref[0:1, sl], jnp.maximum(colmin + b2c, 0.0))
                rowmins.append(jnp.min(prod + b2c, axis=1, keepdims=True))
            rowmin = rowmins[0]
            for rm in rowmins[1:]:
                rowmin = jnp.minimum(rowmin, rm)
            sl_p = pl.ds(p * blk, blk)
            min2_ref[0:1, sl_p] = jnp.minimum(min2_ref[0:1, sl_p],
                                              jnp.maximum(rowmin.T + a2r, 0.0))

    @pl.when((ph == 1) & (p == 0) & (q == 0))
    def _sqrt_real_mins():
        rmin_ref[...] = jnp.sqrt(min2_ref[...])

    @pl.when(ph == 1)
    def _real_gen():
        a = rs_ref[pl.ds(q * blk, blk), :]
        a2 = rncol_ref[pl.ds(q * blk, blk), :]
        rv = rmin_ref[0:1, pl.ds(q * blk, blk)]
        rvt = rv.T
        for ci in range(_NC):
            g = gen_ref[pl.ds(p * blk + ci * cb, cb), :]
            prod = jax.lax.dot_general(a, g, (((1,), (1,)), ((), ())),
                                       preferred_element_type=jnp.float32)
            t = prod + a2
            lk = jnp.min(t, axis=0, keepdims=True)
            cand = jnp.where(t == lk, rvt, _BIG)
            lv = jnp.min(cand, axis=0, keepdims=True)
            sl = pl.ds(p * blk + ci * cb, cb)
            ck = bkey_ref[0:1, sl]
            cv = bval_ref[0:1, sl]
            upd = lk < ck
            bkey_ref[0:1, sl] = jnp.where(upd, lk, ck)
            bval_ref[0:1, sl] = jnp.where(upd, lv, cv)

    @pl.when((ph == 1) & (p == nblk - 1) & (q == nblk - 1))
    def _finalize():
        g2r = gnrow_ref[...]
        d1 = jnp.sqrt(jnp.maximum(bkey_ref[...] + g2r, 0.0))
        d2v = bval_ref[...]
        authen = jax.nn.sigmoid((d2v - d1) * 10.0)
        out_ref[...] = (-100.0 * (jnp.sum(authen) / n)).reshape(1, 1)


@jax.jit
def kernel(real_stats, gen_stats):
    real_stats = jax.lax.stop_gradient(real_stats)
    n, k = real_stats.shape
    assert gen_stats.shape[0] == n
    blk = 1024 if n % 1024 == 0 else max(b for b in (512, 256, 128, 8)
                                         if n % b == 0)
    nblk = n // blk
    out = pl.pallas_call(
        functools.partial(_authpct_body, blk=blk, nblk=nblk, n=n),
        grid=(2, nblk, nblk),
        in_specs=[
            pl.BlockSpec((n, k), lambda ph, p, q: (0, 0)),
            pl.BlockSpec((n, k), lambda ph, p, q: (0, 0)),
        ],
        out_specs=pl.BlockSpec((1, 1), lambda ph, p, q: (0, 0)),
        out_shape=jax.ShapeDtypeStruct((1, 1), jnp.float32),
        scratch_shapes=[
            pltpu.VMEM((n, k), jnp.float32),
            pltpu.VMEM((n, 1), jnp.float32),
            pltpu.VMEM((1, n), jnp.float32),
            pltpu.VMEM((1, n), jnp.float32),
            pltpu.VMEM((1, n), jnp.float32),
            pltpu.VMEM((1, n), jnp.float32),
            pltpu.VMEM((1, n), jnp.float32),
            pltpu.VMEM((1, n), jnp.float32),
        ],
        compiler_params=pltpu.CompilerParams(
            dimension_semantics=("arbitrary", "arbitrary", "arbitrary"),
        ),
    )(real_stats, gen_stats)
    return out[0, 0]


# speedup vs baseline: 1.2163x; 1.2000x over previous
import functools

import jax
import jax.numpy as jnp
from jax.experimental import pallas as pl
from jax.experimental.pallas import tpu as pltpu

_BIG = 1e20
_NC = 4


def _authpct_body(real_ref, gen_ref, out_ref,
                  rs_ref, rncol_ref, rnrow_ref, gnrow_ref,
                  min2_ref, rmin_ref, bkey_ref, bval_ref, *, blk, nblk, n):
    ph = pl.program_id(0)
    p = pl.program_id(1)
    q = pl.program_id(2)

    @pl.when((ph == 0) & (p == 0) & (q == 0))
    def _init():
        r = real_ref[...]
        g = gen_ref[...]
        rs_ref[...] = r * -2.0
        rn = jnp.sum(r * r, axis=1, keepdims=True)
        rncol_ref[...] = rn
        rnrow_ref[...] = rn.T
        gnrow_ref[...] = jnp.sum(g * g, axis=1, keepdims=True).T
        min2_ref[...] = jnp.full((1, n), _BIG, jnp.float32)
        bkey_ref[...] = jnp.full((1, n), _BIG, jnp.float32)
        bval_ref[...] = jnp.zeros((1, n), jnp.float32)

    cb = blk // _NC

    @pl.when((ph == 0) & (p <= q))
    def _real_real():
        a = rs_ref[pl.ds(p * blk, blk), :]
        a2 = rncol_ref[pl.ds(p * blk, blk), :]
        a2r = rnrow_ref[0:1, pl.ds(p * blk, blk)]

        @pl.when(p == q)
        def _diag():
            for ci in range(_NC):
                b = real_ref[pl.ds(q * blk + ci * cb, cb), :]
                prod = jax.lax.dot_general(a, b, (((1,), (1,)), ((), ())),
                                           preferred_element_type=jnp.float32)
                rows = jax.lax.broadcasted_iota(jnp.int32, (blk, cb), 0)
                cols = jax.lax.broadcasted_iota(jnp.int32, (blk, cb), 1)
                masked = jnp.where(rows == cols + ci * cb, _BIG, prod)
                colmin = jnp.min(masked + a2, axis=0, keepdims=True)
                sl = pl.ds(q * blk + ci * cb, cb)
                b2c = rnrow_ref[0:1, sl]
                min2_ref[0:1, sl] = jnp.minimum(
                    min2_ref[0:1, sl], jnp.maximum(colmin + b2c, 0.0))

        @pl.when(p != q)
        def _offdiag():
            rowmins = []
            for ci in range(_NC):
                b = real_ref[pl.ds(q * blk + ci * cb, cb), :]
                prod = jax.lax.dot_general(a, b, (((1,), (1,)), ((), ())),
                                           preferred_element_type=jnp.float32)
                sl = pl.ds(q * blk + ci * cb, cb)
                b2c = rnrow_ref[0:1, sl]
                colmin = jnp.min(prod + a2, axis=0, keepdims=True)
                min2_ref[0:1, sl] = jnp.minimum(
                    min2_ref[0:1, sl], jnp.maximum(colmin + b2c, 0.0))
                rowmins.append(jnp.min(prod + b2c, axis=1, keepdims=True))
            rowmin = rowmins[0]
            for rm in rowmins[1:]:
                rowmin = jnp.minimum(rowmin, rm)
            sl_p = pl.ds(p * blk, blk)
            min2_ref[0:1, sl_p] = jnp.minimum(min2_ref[0:1, sl_p],
                                              jnp.maximum(rowmin.T + a2r, 0.0))

    @pl.when((ph == 1) & (p == 0) & (q == 0))
    def _sqrt_real_mins():
        rmin_ref[...] = jnp.sqrt(min2_ref[...])

    @pl.when(ph == 1)
    def _real_gen():
        a = rs_ref[pl.ds(q * blk, blk), :]
        a2 = rncol_ref[pl.ds(q * blk, blk), :]
        rv = rmin_ref[0:1, pl.ds(q * blk, blk)]
        rvt = rv.T
        for ci in range(_NC):
            g = gen_ref[pl.ds(p * blk + ci * cb, cb), :]
            prod = jax.lax.dot_general(a, g, (((1,), (1,)), ((), ())),
                                       preferred_element_type=jnp.float32)
            t = prod + a2
            lk = jnp.min(t, axis=0, keepdims=True)
            cand = jnp.where(t == lk, rvt, _BIG)
            lv = jnp.min(cand, axis=0, keepdims=True)
            sl = pl.ds(p * blk + ci * cb, cb)
            ck = bkey_ref[0:1, sl]
            cv = bval_ref[0:1, sl]
            upd = lk < ck
            bkey_ref[0:1, sl] = jnp.where(upd, lk, ck)
            bval_ref[0:1, sl] = jnp.where(upd, lv, cv)

    @pl.when((ph == 1) & (p == nblk - 1) & (q == nblk - 1))
    def _finalize():
        g2r = gnrow_ref[...]
        d1 = jnp.sqrt(jnp.maximum(bkey_ref[...] + g2r, 0.0))
        d2v = bval_ref[...]
        authen = jax.nn.sigmoid((d2v - d1) * 10.0)
        out_ref[...] = (-100.0 * (jnp.sum(authen) / n)).reshape(1, 1)


@jax.jit
def kernel(real_stats, gen_stats):
    real_stats = jax.lax.stop_gradient(real_stats)
    n, k = real_stats.shape
    assert gen_stats.shape[0] == n
    blk = 2048 if n % 2048 == 0 else max(b for b in (1024, 512, 256, 128, 8)
                                         if n % b == 0)
    nblk = n // blk
    out = pl.pallas_call(
        functools.partial(_authpct_body, blk=blk, nblk=nblk, n=n),
        grid=(2, nblk, nblk),
        in_specs=[
            pl.BlockSpec((n, k), lambda ph, p, q: (0, 0)),
            pl.BlockSpec((n, k), lambda ph, p, q: (0, 0)),
        ],
        out_specs=pl.BlockSpec((1, 1), lambda ph, p, q: (0, 0)),
        out_shape=jax.ShapeDtypeStruct((1, 1), jnp.float32),
        scratch_shapes=[
            pltpu.VMEM((n, k), jnp.float32),
            pltpu.VMEM((n, 1), jnp.float32),
            pltpu.VMEM((1, n), jnp.float32),
            pltpu.VMEM((1, n), jnp.float32),
            pltpu.VMEM((1, n), jnp.float32),
            pltpu.VMEM((1, n), jnp.float32),
            pltpu.VMEM((1, n), jnp.float32),
            pltpu.VMEM((1, n), jnp.float32),
        ],
        compiler_params=pltpu.CompilerParams(
            dimension_semantics=("arbitrary", "arbitrary", "arbitrary"),
        ),
    )(real_stats, gen_stats)
    return out[0, 0]
